# Initial kernel scaffold; baseline (speedup 1.0000x reference)
import jax
import jax.numpy as jnp
from jax.experimental import pallas as pl


def kernel(x_s, edge_index_s, edge_attr_s, x_s_batch, x_t, edge_index_t, edge_attr_t, x_t_batch, params):
    raise NotImplementedError("write your pallas kernel here")


# speedup vs baseline: 1.3922x; 1.3922x over previous
import jax
import jax.numpy as jnp
from jax.experimental import pallas as pl
from jax.experimental.pallas import tpu as pltpu

N_GRAPHS = 8
_HI = jax.lax.Precision.HIGHEST


def _lrelu(x, s=0.01):
    return jnp.where(x >= 0, x, s * x)


def _bn(x, w, b, eps=1e-5):
    m = x.mean(0)
    v = ((x - m) ** 2).mean(0)
    return (x - m) / jnp.sqrt(v + eps) * w + b


def _bt(x):
    return x.astype(jnp.bfloat16).astype(jnp.float32)


def _pre_e_body(ea_ref, we_ref, be_ref, w1c_ref, bm_ref, out_ref):
    ea = ea_ref[...]
    ee = _lrelu(ea * we_ref[...] + be_ref[...])
    out_ref[...] = jax.lax.dot_general(
        _bt(ee), w1c_ref[...], (((1,), (0,)), ((), ())),
        preferred_element_type=jnp.float32, precision=_HI) + bm_ref[...]


def _pre_e(edge_attr, w_edge_row, b_edge, W1c, b_msg1):
    E = edge_attr.shape[0]
    BLK = 2000
    grid = (E // BLK,)
    return pl.pallas_call(
        _pre_e_body,
        grid=grid,
        in_specs=[
            pl.BlockSpec((BLK, 1), lambda i: (i, 0)),
            pl.BlockSpec((1, 64), lambda i: (0, 0)),
            pl.BlockSpec((1, 64), lambda i: (0, 0)),
            pl.BlockSpec((64, 32), lambda i: (0, 0)),
            pl.BlockSpec((1, 32), lambda i: (0, 0)),
        ],
        out_specs=pl.BlockSpec((BLK, 32), lambda i: (i, 0)),
        out_shape=jax.ShapeDtypeStruct((E, 32), jnp.float32),
    )(edge_attr, w_edge_row, b_edge, _bt(W1c), b_msg1)


def _embed(x, edge_index, edge_attr, batch, P):
    src = edge_index[0]
    dst = edge_index[1]
    n = x.shape[0]
    xe = _lrelu(x @ P['W_node'] + P['b_node'])
    pre_d = xe @ P['W_msg1'][0:64]
    pre_s = xe @ P['W_msg1'][64:128]
    pre_e = _pre_e(edge_attr, P['W_edge'][0:1], P['b_edge'][None, :],
                   P['W_msg1'][128:192], P['b_msg1'][None, :])
    m32 = _lrelu(pre_d[dst] + pre_s[src] + pre_e)
    aggr32 = jnp.zeros((n, 32), jnp.float32).at[dst].add(_bt(m32))
    deg = jnp.zeros((n,), jnp.float32).at[dst].add(1.0)
    aggr = jax.lax.dot_general(aggr32, _bt(P['W_msg2']), (((1,), (0,)), ((), ())),
                               precision=_HI) + deg[:, None] * P['b_msg2']
    h = _lrelu(jnp.concatenate([x, aggr], axis=1) @ P['W_upd'] + P['b_upd'])
    h = _bn(h, P['bn_w'], P['bn_b'])
    hs = h @ P['gat_W']
    a_src = (hs * P['att_src']).sum(-1)
    a_dst = (hs * P['att_dst']).sum(-1)
    ce = (P['gat_We'][0] * P['att_edge']).sum()
    ea = edge_attr[:, 0]
    ea_mean = ea.mean()
    a_e = ea * ce
    a_e_self = ea_mean * ce
    ub = a_src.max() + a_dst.max() + jnp.maximum(jnp.max(a_e), a_e_self)
    S = _lrelu(ub, 0.2)
    alpha_edge = _lrelu(a_src[src] + a_dst[dst] + a_e, 0.2)
    alpha_self = _lrelu(a_src + a_dst + a_e_self, 0.2)
    ex_edge = jnp.exp(alpha_edge - S)
    ex_self = jnp.exp(alpha_self - S)
    den = jnp.zeros((n,), jnp.float32).at[dst].add(ex_edge) + ex_self
    num = jnp.zeros((n, 64), jnp.float32).at[dst].add(ex_edge[:, None] * hs[src]) \
        + ex_self[:, None] * hs
    gat = num / (den[:, None] + 1e-16) + P['gat_b']
    s = gat @ P['ag_W1'] + P['ag_b1']
    s = jnp.where(s >= 0, s, P['prelu_a'] * s)
    g = jax.nn.softmax(gat @ P['ag_Wg'] + P['ag_bg'], axis=1)
    s = _lrelu((s * g) @ P['ag_Wf1'] + P['ag_bf1'])
    s = s @ P['ag_Wf2'] + P['ag_bf2']
    sums = jax.ops.segment_sum(s, batch, num_segments=N_GRAPHS)
    cnt = jax.ops.segment_sum(jnp.ones((n,), dtype=x.dtype), batch, num_segments=N_GRAPHS)
    return sums / jnp.clip(cnt, 1.0)[:, None]


def kernel(x_s, edge_index_s, edge_attr_s, x_s_batch,
           x_t, edge_index_t, edge_attr_t, x_t_batch, params):
    P = params
    es = _embed(x_s, edge_index_s, edge_attr_s, x_s_batch, P)
    et = _embed(x_t, edge_index_t, edge_attr_t, x_t_batch, P)
    out = jnp.concatenate([es, et], axis=1)
    h = out @ P['c_W1'] + P['c_b1']
    h = jax.nn.relu(_bn(h, P['c_bn1w'], P['c_bn1b']))
    h = h @ P['c_W2'] + P['c_b2']
    h = jax.nn.relu(_bn(h, P['c_bn2w'], P['c_bn2b']))
    h = h @ P['c_W3'] + P['c_b3']
    return h


# speedup vs baseline: 10.2473x; 7.3603x over previous
import jax
import jax.numpy as jnp
from jax import lax
from jax.experimental import pallas as pl
from jax.experimental.pallas import tpu as pltpu
from jax.experimental.pallas import tpu_sc as plsc

N_GRAPHS = 8
_HI = jax.lax.Precision.HIGHEST

N_NODES = 10000
N_EDGES = 320000
NPAD = 10240
NSC = 16
PER_W = NPAD // NSC
EPAD = 327680
ES = EPAD // NSC
EC = 512
NCH = ES // EC
SUB = 128
NSUB = EC // SUB
ECG = 512
NCHG = ES // ECG
NSUBG = ECG // SUB


def _lrelu(x, s=0.01):
    return jnp.where(x >= 0, x, s * x)


def _bn(x, w, b, eps=1e-5):
    m = x.mean(0)
    v = ((x - m) ** 2).mean(0)
    return (x - m) / jnp.sqrt(v + eps) * w + b


def _bt(x):
    return x.astype(jnp.bfloat16).astype(jnp.float32)


def _bt_reg(v):
    t = v * 65537.0
    return t - (t - v)


def _pre_e_body(ea_ref, we_ref, be_ref, w1c_ref, bm_ref, out_ref):
    ea = ea_ref[...]
    ee = _lrelu(ea * we_ref[...] + be_ref[...])
    out_ref[...] = jax.lax.dot_general(
        _bt(ee), w1c_ref[...], (((1,), (0,)), ((), ())),
        preferred_element_type=jnp.float32, precision=_HI) + bm_ref[...]


def _pre_e(edge_attr2, w_edge_row, b_edge, W1c, b_msg1):
    E2 = edge_attr2.shape[0]
    BLK = 2000
    grid = (E2 // BLK,)
    return pl.pallas_call(
        _pre_e_body,
        grid=grid,
        in_specs=[
            pl.BlockSpec((BLK, 1), lambda i: (i, 0)),
            pl.BlockSpec((1, 64), lambda i: (0, 0)),
            pl.BlockSpec((1, 64), lambda i: (0, 0)),
            pl.BlockSpec((64, 32), lambda i: (0, 0)),
            pl.BlockSpec((1, 32), lambda i: (0, 0)),
        ],
        out_specs=pl.BlockSpec((BLK, 32), lambda i: (i, 0)),
        out_shape=jax.ShapeDtypeStruct((E2, 32), jnp.float32),
    )(edge_attr2, w_edge_row, b_edge, _bt(W1c), b_msg1)


def _sc_embed_body(pre_d_hbm, pre_s_hbm, src_hbm, dst_hbm, pre_e_hbm,
                   z32_hbm, zn_hbm,
                   aggr_out, deg_out,
                   dstb, gdb, gsb, rows_d, rows_s, rows_e, onesb,
                   aggr_sh, deg_sh, sem):
    c = lax.axis_index("c")
    w = lax.axis_index("s")
    cn16 = jnp.full((16,), c * NPAD, jnp.int32)

    pltpu.sync_copy(z32_hbm, aggr_sh.at[pl.ds(w * PER_W, PER_W)])
    pltpu.sync_copy(zn_hbm, deg_sh.at[pl.ds(w * PER_W, PER_W)])

    def fill_ones(i, _):
        onesb[pl.ds(i * 16, 16)] = jnp.full((16,), 1.0, jnp.float32)
        return 0
    lax.fori_loop(0, SUB // 16, fill_ones, 0)
    plsc.subcore_barrier()

    def chunk(k, _):
        pltpu.sync_copy(dst_hbm.at[c, w, k], dstb)
        pltpu.sync_copy(src_hbm.at[c, w, k], gsb)
        pltpu.sync_copy(pre_e_hbm.at[c, w, k], rows_e)

        def offs(i, _):
            j, q = i // (SUB // 16), i % (SUB // 16)
            sl = pl.ds(q * 16, 16)
            gdb[j, sl] = dstb[j, sl] + cn16
            gsb[j, sl] = gsb[j, sl] + cn16
            return 0
        lax.fori_loop(0, EC // 16, offs, 0)

        cps = []
        for j in range(NSUB):
            cps.append(pltpu.async_copy(pre_d_hbm.at[gdb.at[j]],
                                        rows_d.at[pl.ds(j * SUB, SUB)], sem))
            cps.append(pltpu.async_copy(pre_s_hbm.at[gsb.at[j]],
                                        rows_s.at[pl.ds(j * SUB, SUB)], sem))
        for cp in cps:
            cp.wait()

        def compute(i, _):
            for half in (0, 16):
                sl = pl.ds(half, 16)
                v = rows_d[i, sl] + rows_s[i, sl] + rows_e[i, sl]
                m = jnp.maximum(v, 0.01 * v)
                rows_d[i, sl] = _bt_reg(m)
            return 0
        lax.fori_loop(0, EC, compute, 0)

        for j in range(NSUB):
            pltpu.sync_copy(rows_d.at[pl.ds(j * SUB, SUB)],
                            aggr_sh.at[dstb.at[j]], add=True)
            pltpu.sync_copy(onesb, deg_sh.at[dstb.at[j]], add=True)
        return 0

    lax.fori_loop(0, NCH, chunk, 0)
    plsc.subcore_barrier()

    sl = pl.ds(w * PER_W, PER_W)
    pltpu.sync_copy(aggr_sh.at[sl], aggr_out.at[c, sl])
    pltpu.sync_copy(deg_sh.at[sl], deg_out.at[c, sl])


def _sc_embed(pre_d2, pre_s2, src3, dst3, pre_e3, z32, zn):
    mesh = plsc.VectorSubcoreMesh(core_axis_name="c", subcore_axis_name="s")
    return pl.kernel(
        _sc_embed_body,
        out_type=[
            jax.ShapeDtypeStruct((2, NPAD, 32), jnp.float32),
            jax.ShapeDtypeStruct((2, NPAD), jnp.float32),
        ],
        mesh=mesh,
        compiler_params=pltpu.CompilerParams(use_tc_tiling_on_sc=False, needs_layout_passes=False),
        scratch_types=[
            pltpu.VMEM((NSUB, SUB), jnp.int32),
            pltpu.VMEM((NSUB, SUB), jnp.int32),
            pltpu.VMEM((NSUB, SUB), jnp.int32),
            pltpu.VMEM((EC, 32), jnp.float32),
            pltpu.VMEM((EC, 32), jnp.float32),
            pltpu.VMEM((EC, 32), jnp.float32),
            pltpu.VMEM((SUB,), jnp.float32),
            pltpu.VMEM_SHARED((NPAD, 32), jnp.float32),
            pltpu.VMEM_SHARED((NPAD,), jnp.float32),
            pltpu.SemaphoreType.DMA,
        ],
    )(pre_d2, pre_s2, src3, dst3, pre_e3, z32, zn)


def _sc_gat_body(asrc_hbm, adst_hbm, hs_hbm, src_hbm, dst_hbm, ae_hbm, s_hbm,
                 z64_hbm, zn_hbm,
                 num_out, den_out,
                 asrc_v, adst_v, srcb, dstb, gsb, aeb, exb,
                 hsrows, svec, num_sh, den_sh, sem):
    c = lax.axis_index("c")
    w = lax.axis_index("s")
    cn16 = jnp.full((16,), c * NPAD, jnp.int32)

    pltpu.sync_copy(z64_hbm, num_sh.at[pl.ds(w * PER_W, PER_W)])
    pltpu.sync_copy(zn_hbm, den_sh.at[pl.ds(w * PER_W, PER_W)])
    pltpu.sync_copy(asrc_hbm.at[c], asrc_v)
    pltpu.sync_copy(adst_hbm.at[c], adst_v)
    pltpu.sync_copy(s_hbm.at[c], svec)
    plsc.subcore_barrier()
    sv = svec[...]

    def chunk(k, _):
        pltpu.sync_copy(src_hbm.at[c, w, k], srcb)
        pltpu.sync_copy(dst_hbm.at[c, w, k], dstb)
        pltpu.sync_copy(ae_hbm.at[c, w, k], aeb)

        def offs(i, _):
            j, q = i // (SUB // 16), i % (SUB // 16)
            sl = pl.ds(q * 16, 16)
            gsb[j, sl] = srcb[j, sl] + cn16
            return 0
        lax.fori_loop(0, ECG // 16, offs, 0)

        cps = []
        for j in range(NSUBG):
            cps.append(pltpu.async_copy(hs_hbm.at[gsb.at[j]],
                                        hsrows.at[pl.ds(j * SUB, SUB)], sem))

        def attn(i, _):
            j, q = i // (SUB // 16), i % (SUB // 16)
            sl = pl.ds(q * 16, 16)
            a = plsc.load_gather(asrc_v, [srcb[j, sl]])
            b = plsc.load_gather(adst_v, [dstb[j, sl]])
            t = (a + b) + aeb[j, sl]
            al = jnp.maximum(t, 0.2 * t)
            exb[j, sl] = jnp.exp(al - sv)
            return 0
        lax.fori_loop(0, ECG // 16, attn, 0)

        for cp in cps:
            cp.wait()

        def scale(i, _):
            jj, j = i // SUB, i % SUB
            e16 = plsc.load_gather(exb, [jnp.full((16,), jj, jnp.int32),
                                         jnp.full((16,), j, jnp.int32)])
            for q in range(4):
                sl = pl.ds(q * 16, 16)
                hsrows[i, sl] = hsrows[i, sl] * e16
            return 0
        lax.fori_loop(0, ECG, scale, 0)

        for j in range(NSUBG):
            pltpu.sync_copy(hsrows.at[pl.ds(j * SUB, SUB)],
                            num_sh.at[dstb.at[j]], add=True)
            pltpu.sync_copy(exb.at[j], den_sh.at[dstb.at[j]], add=True)
        return 0

    lax.fori_loop(0, NCHG, chunk, 0)
    plsc.subcore_barrier()

    sl = pl.ds(w * PER_W, PER_W)
    pltpu.sync_copy(num_sh.at[sl], num_out.at[c, sl])
    pltpu.sync_copy(den_sh.at[sl], den_out.at[c, sl])


def _sc_gat(asrc2, adst2, hs2, src3, dst3, ae3, s2, z64, zn):
    mesh = plsc.VectorSubcoreMesh(core_axis_name="c", subcore_axis_name="s")
    return pl.kernel(
        _sc_gat_body,
        out_type=[
            jax.ShapeDtypeStruct((2, NPAD, 64), jnp.float32),
            jax.ShapeDtypeStruct((2, NPAD), jnp.float32),
        ],
        mesh=mesh,
        compiler_params=pltpu.CompilerParams(use_tc_tiling_on_sc=False, needs_layout_passes=False),
        scratch_types=[
            pltpu.VMEM((NPAD,), jnp.float32),
            pltpu.VMEM((NPAD,), jnp.float32),
            pltpu.VMEM((NSUBG, SUB), jnp.int32),
            pltpu.VMEM((NSUBG, SUB), jnp.int32),
            pltpu.VMEM((NSUBG, SUB), jnp.int32),
            pltpu.VMEM((NSUBG, SUB), jnp.float32),
            pltpu.VMEM((NSUBG, SUB), jnp.float32),
            pltpu.VMEM((ECG, 64), jnp.float32),
            pltpu.VMEM((16,), jnp.float32),
            pltpu.VMEM_SHARED((NPAD, 64), jnp.float32),
            pltpu.VMEM_SHARED((NPAD,), jnp.float32),
            pltpu.SemaphoreType.DMA,
        ],
    )(asrc2, adst2, hs2, src3, dst3, ae3, s2, z64, zn)


def _pad_edges(idx):
    return jnp.concatenate(
        [idx, jnp.full((EPAD - N_EDGES,), NPAD - 1, jnp.int32)])


def kernel(x_s, edge_index_s, edge_attr_s, x_s_batch,
           x_t, edge_index_t, edge_attr_t, x_t_batch, params):
    P = params
    n = x_s.shape[0]

    xe_s = _lrelu(x_s @ P['W_node'] + P['b_node'])
    xe_t = _lrelu(x_t @ P['W_node'] + P['b_node'])
    pre_d_s = xe_s @ P['W_msg1'][0:64]
    pre_s_s = xe_s @ P['W_msg1'][64:128]
    pre_d_t = xe_t @ P['W_msg1'][0:64]
    pre_s_t = xe_t @ P['W_msg1'][64:128]

    def padn(a):
        return jnp.pad(a, ((0, NPAD - n), (0, 0)))

    pre_d2 = jnp.concatenate([padn(pre_d_s), padn(pre_d_t)])
    pre_s2 = jnp.concatenate([padn(pre_s_s), padn(pre_s_t)])

    src3 = jnp.stack([_pad_edges(edge_index_s[0]),
                      _pad_edges(edge_index_t[0])]).reshape(2, NSC, NCH, NSUB, SUB)
    dst3 = jnp.stack([_pad_edges(edge_index_s[1]),
                      _pad_edges(edge_index_t[1])]).reshape(2, NSC, NCH, NSUB, SUB)

    ea_cat = jnp.concatenate([edge_attr_s, edge_attr_t])
    pre_e = _pre_e(ea_cat, P['W_edge'][0:1], P['b_edge'][None, :],
                   P['W_msg1'][128:192], P['b_msg1'][None, :])
    pre_e3 = jnp.pad(pre_e.reshape(2, N_EDGES, 32),
                     ((0, 0), (0, EPAD - N_EDGES), (0, 0))
                     ).reshape(2, NSC, NCH, EC, 32)

    z32 = jnp.zeros((PER_W, 32), jnp.float32)
    z64 = jnp.zeros((PER_W, 64), jnp.float32)
    zn = jnp.zeros((PER_W,), jnp.float32)

    aggr32, deg = _sc_embed(pre_d2, pre_s2, src3, dst3, pre_e3, z32, zn)

    W2b = _bt(P['W_msg2'])
    outs = []
    svals = []
    gat_inputs = []
    for g, (x, ea) in enumerate(((x_s, edge_attr_s), (x_t, edge_attr_t))):
        aggr = jax.lax.dot_general(aggr32[g, :n], W2b, (((1,), (0,)), ((), ())),
                                   precision=_HI) + deg[g, :n, None] * P['b_msg2']
        h = _lrelu(jnp.concatenate([x, aggr], axis=1) @ P['W_upd'] + P['b_upd'])
        h = _bn(h, P['bn_w'], P['bn_b'])
        hs = h @ P['gat_W']
        a_src = (hs * P['att_src']).sum(-1)
        a_dst = (hs * P['att_dst']).sum(-1)
        ce = (P['gat_We'][0] * P['att_edge']).sum()
        eav = ea[:, 0]
        ea_mean = eav.mean()
        a_e = eav * ce
        a_e_self = ea_mean * ce
        ub = a_src.max() + a_dst.max() + jnp.maximum(jnp.max(a_e), a_e_self)
        S = _lrelu(ub, 0.2)
        gat_inputs.append((hs, a_src, a_dst, a_e, a_e_self, S))

    def padv(a):
        return jnp.pad(a, ((0, NPAD - n),))

    asrc2 = jnp.stack([padv(gat_inputs[0][1]), padv(gat_inputs[1][1])])
    adst2 = jnp.stack([padv(gat_inputs[0][2]), padv(gat_inputs[1][2])])
    hs2 = jnp.concatenate([padn(gat_inputs[0][0]), padn(gat_inputs[1][0])])
    ae3 = jnp.stack([
        jnp.pad(gat_inputs[0][3], (0, EPAD - N_EDGES)),
        jnp.pad(gat_inputs[1][3], (0, EPAD - N_EDGES)),
    ]).reshape(2, NSC, NCHG, NSUBG, SUB)
    s2 = jnp.stack([jnp.broadcast_to(gat_inputs[0][5], (16,)),
                    jnp.broadcast_to(gat_inputs[1][5], (16,))])

    num, den = _sc_gat(asrc2, adst2, hs2, src3, dst3, ae3, s2, z64, zn)

    for g, batch in enumerate((x_s_batch, x_t_batch)):
        hs, a_src, a_dst, a_e, a_e_self, S = gat_inputs[g]
        alpha_self = _lrelu(a_src + a_dst + a_e_self, 0.2)
        ex_self = jnp.exp(alpha_self - S)
        den_g = den[g, :n] + ex_self
        num_g = num[g, :n] + ex_self[:, None] * hs
        gat = num_g / (den_g[:, None] + 1e-16) + P['gat_b']
        s = gat @ P['ag_W1'] + P['ag_b1']
        s = jnp.where(s >= 0, s, P['prelu_a'] * s)
        gsm = jax.nn.softmax(gat @ P['ag_Wg'] + P['ag_bg'], axis=1)
        s = _lrelu((s * gsm) @ P['ag_Wf1'] + P['ag_bf1'])
        s = s @ P['ag_Wf2'] + P['ag_bf2']
        oh = (batch[None, :] == jnp.arange(N_GRAPHS, dtype=jnp.int32)[:, None]
              ).astype(jnp.float32)
        sums = jax.lax.dot_general(oh, s, (((1,), (0,)), ((), ())),
                                   precision=_HI)
        outs.append(sums / jnp.clip(oh.sum(1), 1.0)[:, None])

    out = jnp.concatenate(outs, axis=1)
    h = out @ P['c_W1'] + P['c_b1']
    h = jax.nn.relu(_bn(h, P['c_bn1w'], P['c_bn1b']))
    h = h @ P['c_W2'] + P['c_b2']
    h = jax.nn.relu(_bn(h, P['c_bn2w'], P['c_bn2b']))
    h = h @ P['c_W3'] + P['c_b3']
    return h


# speedup vs baseline: 10.6858x; 1.0428x over previous
import jax
import jax.numpy as jnp
from jax import lax
from jax.experimental import pallas as pl
from jax.experimental.pallas import tpu as pltpu
from jax.experimental.pallas import tpu_sc as plsc

N_GRAPHS = 8
_HI = jax.lax.Precision.HIGHEST

N_NODES = 10000
N_EDGES = 320000
NPAD = 10240
NSC = 16
PER_W = NPAD // NSC
EPAD = 327680
ES = EPAD // NSC
EC = 512
NCH = ES // EC
SUB = 128
NSUB = EC // SUB
ECG = 512
NCHG = ES // ECG
NSUBG = ECG // SUB


def _lrelu(x, s=0.01):
    return jnp.where(x >= 0, x, s * x)


def _bn(x, w, b, eps=1e-5):
    m = x.mean(0)
    v = ((x - m) ** 2).mean(0)
    return (x - m) / jnp.sqrt(v + eps) * w + b


def _bt(x):
    return x.astype(jnp.bfloat16).astype(jnp.float32)


def _bt_reg(v):
    t = v * 65537.0
    return t - (t - v)


def _pre_e_body(ea_ref, we_ref, be_ref, w1c_ref, bm_ref, out_ref):
    ea = ea_ref[...]
    ee = _lrelu(ea * we_ref[...] + be_ref[...])
    out_ref[...] = jax.lax.dot_general(
        _bt(ee), w1c_ref[...], (((1,), (0,)), ((), ())),
        preferred_element_type=jnp.float32, precision=_HI) + bm_ref[...]


def _pre_e(edge_attr2, w_edge_row, b_edge, W1c, b_msg1):
    E2 = edge_attr2.shape[0]
    BLK = 2000
    grid = (E2 // BLK,)
    return pl.pallas_call(
        _pre_e_body,
        grid=grid,
        in_specs=[
            pl.BlockSpec((BLK, 1), lambda i: (i, 0)),
            pl.BlockSpec((1, 64), lambda i: (0, 0)),
            pl.BlockSpec((1, 64), lambda i: (0, 0)),
            pl.BlockSpec((64, 32), lambda i: (0, 0)),
            pl.BlockSpec((1, 32), lambda i: (0, 0)),
        ],
        out_specs=pl.BlockSpec((BLK, 32), lambda i: (i, 0)),
        out_shape=jax.ShapeDtypeStruct((E2, 32), jnp.float32),
    )(edge_attr2, w_edge_row, b_edge, _bt(W1c), b_msg1)


def _sc_embed_body(pre_d_hbm, pre_s_hbm, src_hbm, dst_hbm, pre_e_hbm,
                   z32_hbm, zn_hbm,
                   aggr_out, deg_out,
                   dstb, gdb, gsb, rows_d, rows_s, rows_e, onesb,
                   aggr_sh, deg_sh, sem):
    c = lax.axis_index("c")
    w = lax.axis_index("s")
    cn16 = jnp.full((16,), c * NPAD, jnp.int32)

    pltpu.sync_copy(z32_hbm, aggr_sh.at[pl.ds(w * PER_W, PER_W)])
    pltpu.sync_copy(zn_hbm, deg_sh.at[pl.ds(w * PER_W, PER_W)])

    def fill_ones(i, _):
        onesb[pl.ds(i * 16, 16)] = jnp.full((16,), 1.0, jnp.float32)
        return 0
    lax.fori_loop(0, SUB // 16, fill_ones, 0)
    plsc.subcore_barrier()

    def chunk(k, _):
        lds = [pltpu.async_copy(dst_hbm.at[c, w, k], dstb, sem),
               pltpu.async_copy(src_hbm.at[c, w, k], gsb, sem),
               pltpu.async_copy(pre_e_hbm.at[c, w, k], rows_e, sem)]
        for cp in lds:
            cp.wait()

        def offs(i, _):
            j, q = i // (SUB // 16), i % (SUB // 16)
            sl = pl.ds(q * 16, 16)
            gdb[j, sl] = dstb[j, sl] + cn16
            gsb[j, sl] = gsb[j, sl] + cn16
            return 0
        lax.fori_loop(0, EC // 16, offs, 0)

        cps = []
        for j in range(NSUB):
            cps.append(pltpu.async_copy(pre_d_hbm.at[gdb.at[j]],
                                        rows_d.at[pl.ds(j * SUB, SUB)], sem))
            cps.append(pltpu.async_copy(pre_s_hbm.at[gsb.at[j]],
                                        rows_s.at[pl.ds(j * SUB, SUB)], sem))
        for cp in cps:
            cp.wait()

        def compute(i, _):
            for half in (0, 16):
                sl = pl.ds(half, 16)
                v = rows_d[i, sl] + rows_s[i, sl] + rows_e[i, sl]
                m = jnp.maximum(v, 0.01 * v)
                rows_d[i, sl] = _bt_reg(m)
            return 0
        lax.fori_loop(0, EC, compute, 0)

        sc = []
        for j in range(NSUB):
            sc.append(pltpu.async_copy(rows_d.at[pl.ds(j * SUB, SUB)],
                                       aggr_sh.at[dstb.at[j]], sem, add=True))
            sc.append(pltpu.async_copy(onesb, deg_sh.at[dstb.at[j]], sem,
                                       add=True))
        for cp in sc:
            cp.wait()
        return 0

    lax.fori_loop(0, NCH, chunk, 0)
    plsc.subcore_barrier()

    sl = pl.ds(w * PER_W, PER_W)
    pltpu.sync_copy(aggr_sh.at[sl], aggr_out.at[c, sl])
    pltpu.sync_copy(deg_sh.at[sl], deg_out.at[c, sl])


def _sc_embed(pre_d2, pre_s2, src3, dst3, pre_e3, z32, zn):
    mesh = plsc.VectorSubcoreMesh(core_axis_name="c", subcore_axis_name="s")
    return pl.kernel(
        _sc_embed_body,
        out_type=[
            jax.ShapeDtypeStruct((2, NPAD, 32), jnp.float32),
            jax.ShapeDtypeStruct((2, NPAD), jnp.float32),
        ],
        mesh=mesh,
        compiler_params=pltpu.CompilerParams(use_tc_tiling_on_sc=False, needs_layout_passes=False),
        scratch_types=[
            pltpu.VMEM((NSUB, SUB), jnp.int32),
            pltpu.VMEM((NSUB, SUB), jnp.int32),
            pltpu.VMEM((NSUB, SUB), jnp.int32),
            pltpu.VMEM((EC, 32), jnp.float32),
            pltpu.VMEM((EC, 32), jnp.float32),
            pltpu.VMEM((EC, 32), jnp.float32),
            pltpu.VMEM((SUB,), jnp.float32),
            pltpu.VMEM_SHARED((NPAD, 32), jnp.float32),
            pltpu.VMEM_SHARED((NPAD,), jnp.float32),
            pltpu.SemaphoreType.DMA,
        ],
    )(pre_d2, pre_s2, src3, dst3, pre_e3, z32, zn)


def _sc_gat_body(asrc_hbm, adst_hbm, hs_hbm, src_hbm, dst_hbm, ae_hbm, s_hbm,
                 z64_hbm, zn_hbm,
                 num_out, den_out,
                 asrc_v, adst_v, srcb, dstb, gsb, aeb, exb,
                 hsrows, svec, num_sh, den_sh, sem):
    c = lax.axis_index("c")
    w = lax.axis_index("s")
    cn16 = jnp.full((16,), c * NPAD, jnp.int32)

    pltpu.sync_copy(z64_hbm, num_sh.at[pl.ds(w * PER_W, PER_W)])
    pltpu.sync_copy(zn_hbm, den_sh.at[pl.ds(w * PER_W, PER_W)])
    pltpu.sync_copy(asrc_hbm.at[c], asrc_v)
    pltpu.sync_copy(adst_hbm.at[c], adst_v)
    pltpu.sync_copy(s_hbm.at[c], svec)
    plsc.subcore_barrier()
    sv = svec[...]

    def chunk(k, _):
        lds = [pltpu.async_copy(src_hbm.at[c, w, k], srcb, sem),
               pltpu.async_copy(dst_hbm.at[c, w, k], dstb, sem),
               pltpu.async_copy(ae_hbm.at[c, w, k], aeb, sem)]
        for cp in lds:
            cp.wait()

        def offs(i, _):
            j, q = i // (SUB // 16), i % (SUB // 16)
            sl = pl.ds(q * 16, 16)
            gsb[j, sl] = srcb[j, sl] + cn16
            return 0
        lax.fori_loop(0, ECG // 16, offs, 0)

        cps = []
        for j in range(NSUBG):
            cps.append(pltpu.async_copy(hs_hbm.at[gsb.at[j]],
                                        hsrows.at[pl.ds(j * SUB, SUB)], sem))

        def attn(i, _):
            j, q = i // (SUB // 16), i % (SUB // 16)
            sl = pl.ds(q * 16, 16)
            a = plsc.load_gather(asrc_v, [srcb[j, sl]])
            b = plsc.load_gather(adst_v, [dstb[j, sl]])
            t = (a + b) + aeb[j, sl]
            al = jnp.maximum(t, 0.2 * t)
            exb[j, sl] = jnp.exp(al - sv)
            return 0
        lax.fori_loop(0, ECG // 16, attn, 0)

        for cp in cps:
            cp.wait()

        def scale(i, _):
            jj, j = i // SUB, i % SUB
            e16 = plsc.load_gather(exb, [jnp.full((16,), jj, jnp.int32),
                                         jnp.full((16,), j, jnp.int32)])
            for q in range(4):
                sl = pl.ds(q * 16, 16)
                hsrows[i, sl] = hsrows[i, sl] * e16
            return 0
        lax.fori_loop(0, ECG, scale, 0)

        sc = []
        for j in range(NSUBG):
            sc.append(pltpu.async_copy(hsrows.at[pl.ds(j * SUB, SUB)],
                                       num_sh.at[dstb.at[j]], sem, add=True))
            sc.append(pltpu.async_copy(exb.at[j], den_sh.at[dstb.at[j]], sem,
                                       add=True))
        for cp in sc:
            cp.wait()
        return 0

    lax.fori_loop(0, NCHG, chunk, 0)
    plsc.subcore_barrier()

    sl = pl.ds(w * PER_W, PER_W)
    pltpu.sync_copy(num_sh.at[sl], num_out.at[c, sl])
    pltpu.sync_copy(den_sh.at[sl], den_out.at[c, sl])


def _sc_gat(asrc2, adst2, hs2, src3, dst3, ae3, s2, z64, zn):
    mesh = plsc.VectorSubcoreMesh(core_axis_name="c", subcore_axis_name="s")
    return pl.kernel(
        _sc_gat_body,
        out_type=[
            jax.ShapeDtypeStruct((2, NPAD, 64), jnp.float32),
            jax.ShapeDtypeStruct((2, NPAD), jnp.float32),
        ],
        mesh=mesh,
        compiler_params=pltpu.CompilerParams(use_tc_tiling_on_sc=False, needs_layout_passes=False),
        scratch_types=[
            pltpu.VMEM((NPAD,), jnp.float32),
            pltpu.VMEM((NPAD,), jnp.float32),
            pltpu.VMEM((NSUBG, SUB), jnp.int32),
            pltpu.VMEM((NSUBG, SUB), jnp.int32),
            pltpu.VMEM((NSUBG, SUB), jnp.int32),
            pltpu.VMEM((NSUBG, SUB), jnp.float32),
            pltpu.VMEM((NSUBG, SUB), jnp.float32),
            pltpu.VMEM((ECG, 64), jnp.float32),
            pltpu.VMEM((16,), jnp.float32),
            pltpu.VMEM_SHARED((NPAD, 64), jnp.float32),
            pltpu.VMEM_SHARED((NPAD,), jnp.float32),
            pltpu.SemaphoreType.DMA,
        ],
    )(asrc2, adst2, hs2, src3, dst3, ae3, s2, z64, zn)


def _pad_edges(idx):
    return jnp.concatenate(
        [idx, jnp.full((EPAD - N_EDGES,), NPAD - 1, jnp.int32)])


def kernel(x_s, edge_index_s, edge_attr_s, x_s_batch,
           x_t, edge_index_t, edge_attr_t, x_t_batch, params):
    P = params
    n = x_s.shape[0]

    xe_s = _lrelu(x_s @ P['W_node'] + P['b_node'])
    xe_t = _lrelu(x_t @ P['W_node'] + P['b_node'])
    pre_d_s = xe_s @ P['W_msg1'][0:64]
    pre_s_s = xe_s @ P['W_msg1'][64:128]
    pre_d_t = xe_t @ P['W_msg1'][0:64]
    pre_s_t = xe_t @ P['W_msg1'][64:128]

    def padn(a):
        return jnp.pad(a, ((0, NPAD - n), (0, 0)))

    pre_d2 = jnp.concatenate([padn(pre_d_s), padn(pre_d_t)])
    pre_s2 = jnp.concatenate([padn(pre_s_s), padn(pre_s_t)])

    src3 = jnp.stack([_pad_edges(edge_index_s[0]),
                      _pad_edges(edge_index_t[0])]).reshape(2, NSC, NCH, NSUB, SUB)
    dst3 = jnp.stack([_pad_edges(edge_index_s[1]),
                      _pad_edges(edge_index_t[1])]).reshape(2, NSC, NCH, NSUB, SUB)

    ea_cat = jnp.concatenate([edge_attr_s, edge_attr_t])
    pre_e = _pre_e(ea_cat, P['W_edge'][0:1], P['b_edge'][None, :],
                   P['W_msg1'][128:192], P['b_msg1'][None, :])
    pre_e3 = jnp.pad(pre_e.reshape(2, N_EDGES, 32),
                     ((0, 0), (0, EPAD - N_EDGES), (0, 0))
                     ).reshape(2, NSC, NCH, EC, 32)

    z32 = jnp.zeros((PER_W, 32), jnp.float32)
    z64 = jnp.zeros((PER_W, 64), jnp.float32)
    zn = jnp.zeros((PER_W,), jnp.float32)

    aggr32, deg = _sc_embed(pre_d2, pre_s2, src3, dst3, pre_e3, z32, zn)

    W2b = _bt(P['W_msg2'])
    outs = []
    svals = []
    gat_inputs = []
    for g, (x, ea) in enumerate(((x_s, edge_attr_s), (x_t, edge_attr_t))):
        aggr = jax.lax.dot_general(aggr32[g, :n], W2b, (((1,), (0,)), ((), ())),
                                   precision=_HI) + deg[g, :n, None] * P['b_msg2']
        h = _lrelu(jnp.concatenate([x, aggr], axis=1) @ P['W_upd'] + P['b_upd'])
        h = _bn(h, P['bn_w'], P['bn_b'])
        hs = h @ P['gat_W']
        a_src = (hs * P['att_src']).sum(-1)
        a_dst = (hs * P['att_dst']).sum(-1)
        ce = (P['gat_We'][0] * P['att_edge']).sum()
        eav = ea[:, 0]
        ea_mean = eav.mean()
        a_e = eav * ce
        a_e_self = ea_mean * ce
        ub = a_src.max() + a_dst.max() + jnp.maximum(jnp.max(a_e), a_e_self)
        S = _lrelu(ub, 0.2)
        gat_inputs.append((hs, a_src, a_dst, a_e, a_e_self, S))

    def padv(a):
        return jnp.pad(a, ((0, NPAD - n),))

    asrc2 = jnp.stack([padv(gat_inputs[0][1]), padv(gat_inputs[1][1])])
    adst2 = jnp.stack([padv(gat_inputs[0][2]), padv(gat_inputs[1][2])])
    hs2 = jnp.concatenate([padn(gat_inputs[0][0]), padn(gat_inputs[1][0])])
    ae3 = jnp.stack([
        jnp.pad(gat_inputs[0][3], (0, EPAD - N_EDGES)),
        jnp.pad(gat_inputs[1][3], (0, EPAD - N_EDGES)),
    ]).reshape(2, NSC, NCHG, NSUBG, SUB)
    s2 = jnp.stack([jnp.broadcast_to(gat_inputs[0][5], (16,)),
                    jnp.broadcast_to(gat_inputs[1][5], (16,))])

    num, den = _sc_gat(asrc2, adst2, hs2, src3, dst3, ae3, s2, z64, zn)

    for g, batch in enumerate((x_s_batch, x_t_batch)):
        hs, a_src, a_dst, a_e, a_e_self, S = gat_inputs[g]
        alpha_self = _lrelu(a_src + a_dst + a_e_self, 0.2)
        ex_self = jnp.exp(alpha_self - S)
        den_g = den[g, :n] + ex_self
        num_g = num[g, :n] + ex_self[:, None] * hs
        gat = num_g / (den_g[:, None] + 1e-16) + P['gat_b']
        s = gat @ P['ag_W1'] + P['ag_b1']
        s = jnp.where(s >= 0, s, P['prelu_a'] * s)
        gsm = jax.nn.softmax(gat @ P['ag_Wg'] + P['ag_bg'], axis=1)
        s = _lrelu((s * gsm) @ P['ag_Wf1'] + P['ag_bf1'])
        s = s @ P['ag_Wf2'] + P['ag_bf2']
        oh = (batch[None, :] == jnp.arange(N_GRAPHS, dtype=jnp.int32)[:, None]
              ).astype(jnp.float32)
        sums = jax.lax.dot_general(oh, s, (((1,), (0,)), ((), ())),
                                   precision=_HI)
        outs.append(sums / jnp.clip(oh.sum(1), 1.0)[:, None])

    out = jnp.concatenate(outs, axis=1)
    h = out @ P['c_W1'] + P['c_b1']
    h = jax.nn.relu(_bn(h, P['c_bn1w'], P['c_bn1b']))
    h = h @ P['c_W2'] + P['c_b2']
    h = jax.nn.relu(_bn(h, P['c_bn2w'], P['c_bn2b']))
    h = h @ P['c_W3'] + P['c_b3']
    return h
